# baseline (speedup 1.0000x reference)
import functools
import numpy as np
import jax
import jax.numpy as jnp
from jax.experimental import pallas as pl
from jax.experimental.pallas import tpu as pltpu
from jax.experimental.pallas import tpu_sc as plsc

_LATENT = 128
_NODE_DIM = 32
_EDGE_DIM = 16
_HID = 32
_MAXN = 9
_HEADS = 4

_G = 128
_CG = 16
_NC = _G // _CG
_NB = _CG * _MAXN

_WEIGHT_NAMES = (
    'nn_W1', 'nn_b1', 'nn_W2', 'nn_b2', 'nn_W3', 'nn_b3',
    'gen_W1', 'gen_b1', 'gen_W2', 'gen_b2', 'gen_W3', 'gen_b3',
    'Wn0', 'Wn1', 'Wn2', 'Wn3', 'Wn4', 'Wn5', 'Wn6', 'Wn7', 'Wn8',
    'bn',
    'Wqkv0', 'Wqkv1',
    'f_W1', 'f_W2', 'f_W3',
    'maskf',
)


def _act(x):
    return jnp.where(x >= 0, x, 0.01 * x)


def _attn_chunk(qkv, maskf):
    acc = None
    for hd in range(_HEADS):
        sq = slice(hd * _HID, (hd + 1) * _HID)
        sk = slice(128 + hd * _HID, 128 + (hd + 1) * _HID)
        sv = slice(256 + hd * _HID, 256 + (hd + 1) * _HID)
        logits = jax.lax.dot_general(
            qkv[:, sq], qkv[:, sk], (((1,), (1,)), ((), ())),
            preferred_element_type=jnp.float32)
        ex = jnp.exp(logits) * maskf
        den = jnp.sum(ex, axis=1, keepdims=True)
        o = (ex @ qkv[:, sv]) / den
        acc = o if acc is None else acc + o
    return acc


def _fused_kernel(lat_ref, *refs):
    w = dict(zip(_WEIGHT_NAMES, refs[:len(_WEIGHT_NAMES)]))
    emb_ref, nn_ref, sa_ref, sb_ref = refs[len(_WEIGHT_NAMES):]
    lat = lat_ref[...]
    maskf = w['maskf'][...]

    h = _act(lat @ w['nn_W1'][...] + w['nn_b1'][...])
    h = _act(h @ w['nn_W2'][...] + w['nn_b2'][...])
    nn_ref[...] = h @ w['nn_W3'][...] + w['nn_b3'][...]

    g = jnp.tanh(lat @ w['gen_W1'][...] + w['gen_b1'][...])
    g = jnp.tanh(g @ w['gen_W2'][...] + w['gen_b2'][...])
    g = jnp.tanh(g @ w['gen_W3'][...] + w['gen_b3'][...])
    bn = w['bn'][...]
    for n in range(_MAXN):
        p_n = g @ w['Wn%d' % n][...] + bn[n:n + 1, :]
        for c in range(_NC):
            sa_ref[c * _NB + n * _CG:c * _NB + (n + 1) * _CG, :] = \
                p_n[c * _CG:(c + 1) * _CG, :]

    for src, dst, l in ((sa_ref, sb_ref, 0), (sb_ref, sa_ref, 1)):
        wqkv = w['Wqkv%d' % l][...]
        for c2 in range(_NC // 2):
            qkv2 = src[2 * c2 * _NB:(2 * c2 + 2) * _NB, :] @ wqkv
            for half in range(2):
                c = 2 * c2 + half
                dst[c * _NB:(c + 1) * _NB, :] = _attn_chunk(
                    qkv2[half * _NB:(half + 1) * _NB, :], maskf)

    wqkv = w['Wqkv1'][...]
    for c2 in range(_NC // 2):
        qkv2 = sa_ref[2 * c2 * _NB:(2 * c2 + 2) * _NB, :] @ wqkv
        for half in range(2):
            c = 2 * c2 + half
            emb = _attn_chunk(qkv2[half * _NB:(half + 1) * _NB, :], maskf)
            emb = _act(emb @ w['f_W1'][...])
            emb = _act(emb @ w['f_W2'][...])
            emb = emb @ w['f_W3'][...]
            for n in range(_MAXN):
                emb_ref[pl.Slice(c * _NB + n, _CG, _MAXN), :] = \
                    emb[n * _CG:(n + 1) * _CG, :]


def _edge_index_np(nb):
    i = np.arange(_MAXN)
    s0, d0 = np.meshgrid(i, i, indexing='ij')
    s0 = s0.reshape(-1)
    d0 = d0.reshape(-1)
    m = s0 != d0
    s0 = s0[m]
    d0 = d0[m]
    off = (np.arange(nb) * _MAXN)[:, None]
    src = (off + s0[None, :]).reshape(-1)
    dst = (off + d0[None, :]).reshape(-1)
    return np.stack([src, dst]).astype(np.int32)


def _attr_zeros_sc(e):
    info = plsc.get_sparse_core_info()
    nw = info.num_cores * info.num_subcores
    rows = e // nw
    buf_rows = 512
    reps = rows // buf_rows
    mesh = plsc.VectorSubcoreMesh(core_axis_name="c", subcore_axis_name="s")

    @functools.partial(
        pl.kernel, mesh=mesh,
        out_type=jax.ShapeDtypeStruct((e, _EDGE_DIM), jnp.float32),
        scratch_types=[pltpu.VMEM((buf_rows, _EDGE_DIM), jnp.float32)])
    def k(out_hbm, buf):
        wid = (jax.lax.axis_index("s") * info.num_cores
               + jax.lax.axis_index("c"))
        base = wid * rows
        zero = jnp.zeros((_EDGE_DIM,), jnp.float32)
        for i in range(buf_rows):
            buf[i, :] = zero
        for j in range(reps):
            pltpu.sync_copy(buf, out_hbm.at[pl.ds(base + j * buf_rows,
                                                  buf_rows)])

    return k()


def kernel(latent_vec, batch, params):
    nb = latent_vec.shape[0]
    n = nb * _MAXN
    grid = nb // _G
    scale = 1.0 / np.sqrt(float(_HID))

    p = dict(params)
    wp = {k: p[k] for k in ('nn_W1', 'nn_W2', 'nn_W3',
                            'gen_W1', 'gen_W2', 'gen_W3',
                            'f_W2', 'f_W3')}
    for k in ('nn_b1', 'nn_b2', 'nn_b3', 'gen_b1', 'gen_b2', 'gen_b3'):
        wp[k] = p[k].reshape(1, -1)
    for nn_ in range(_MAXN):
        wp['Wn%d' % nn_] = p['gen_Wn'][:, nn_ * _HID:(nn_ + 1) * _HID]
    wp['bn'] = p['gen_bn'].reshape(_MAXN, _HID)
    wp['Wqkv0'] = jnp.concatenate(
        [p['Wq0'] * scale, p['Wk0'], p['Wv0']], axis=1)
    wp['Wqkv1'] = jnp.concatenate(
        [p['Wq1'] * (scale / _HEADS), p['Wk1'] / _HEADS,
         p['Wv1'] / _HEADS], axis=1)
    wp['f_W1'] = p['f_W1'] / _HEADS
    r = np.arange(_NB)
    wp['maskf'] = jnp.asarray(
        (((r[:, None] % _CG) == (r[None, :] % _CG))
         & (r[:, None] != r[None, :])).astype(np.float32))

    ws = [wp[k] for k in _WEIGHT_NAMES]
    in_specs = [pl.BlockSpec((_G, _LATENT), lambda i: (i, 0))]
    for a in ws:
        in_specs.append(pl.BlockSpec(a.shape, lambda i: (0,) * a.ndim))

    emb, nn = pl.pallas_call(
        _fused_kernel,
        grid=(grid,),
        in_specs=in_specs,
        out_specs=[
            pl.BlockSpec((_G * _MAXN, _HID), lambda i: (i, 0)),
            pl.BlockSpec((_G, 1), lambda i: (i, 0)),
        ],
        out_shape=[
            jax.ShapeDtypeStruct((n, _HID), jnp.float32),
            jax.ShapeDtypeStruct((nb, 1), jnp.float32),
        ],
        scratch_shapes=[
            pltpu.VMEM((_NC * _NB, _HID), jnp.float32),
            pltpu.VMEM((_NC * _NB, _HID), jnp.float32),
        ],
    )(latent_vec, *ws)

    edge_index = jnp.asarray(_edge_index_np(nb))
    attr = _attr_zeros_sc(edge_index.shape[1])
    return emb, edge_index, attr, batch, nn.reshape(-1)


# speedup vs baseline: 1.3066x; 1.3066x over previous
import numpy as np
import jax
import jax.numpy as jnp
from jax.experimental import pallas as pl
from jax.experimental.pallas import tpu as pltpu

_LATENT = 128
_NODE_DIM = 32
_EDGE_DIM = 16
_HID = 32
_MAXN = 9
_HEADS = 4

_G = 128
_CG = 16
_NC = _G // _CG
_NB = _CG * _MAXN

_WEIGHT_NAMES = (
    'nn_W1', 'nn_b1', 'nn_W2', 'nn_b2', 'nn_W3', 'nn_b3',
    'gen_W1', 'gen_b1', 'gen_W2', 'gen_b2', 'gen_W3', 'gen_b3',
    'Wn0', 'Wn1', 'Wn2', 'Wn3', 'Wn4', 'Wn5', 'Wn6', 'Wn7', 'Wn8',
    'bn',
    'Wqkv0', 'Wqkv1',
    'f_W1', 'f_W2', 'f_W3',
    'maskf',
)


def _act(x):
    return jnp.where(x >= 0, x, 0.01 * x)


def _attn_chunk(qkv, maskf):
    acc = None
    for hd in range(_HEADS):
        sq = slice(hd * _HID, (hd + 1) * _HID)
        sk = slice(128 + hd * _HID, 128 + (hd + 1) * _HID)
        sv = slice(256 + hd * _HID, 256 + (hd + 1) * _HID)
        logits = jax.lax.dot_general(
            qkv[:, sq], qkv[:, sk], (((1,), (1,)), ((), ())),
            preferred_element_type=jnp.float32)
        ex = jnp.exp(logits) * maskf
        den = jnp.sum(ex, axis=1, keepdims=True)
        o = jax.lax.dot_general(
            ex.astype(jnp.bfloat16), qkv[:, sv], (((1,), (0,)), ((), ())),
            preferred_element_type=jnp.float32) / den
        acc = o if acc is None else acc + o
    return acc


def _fused_kernel(lat_ref, *refs):
    w = dict(zip(_WEIGHT_NAMES, refs[:len(_WEIGHT_NAMES)]))
    emb_ref, nn_ref, sa_ref, sb_ref = refs[len(_WEIGHT_NAMES):]
    lat = lat_ref[...]
    maskf = w['maskf'][...]

    h = _act(lat @ w['nn_W1'][...] + w['nn_b1'][...])
    h = _act(h @ w['nn_W2'][...] + w['nn_b2'][...])
    nn_ref[...] = h @ w['nn_W3'][...] + w['nn_b3'][...]

    g = jnp.tanh(lat @ w['gen_W1'][...] + w['gen_b1'][...])
    g = jnp.tanh(g @ w['gen_W2'][...] + w['gen_b2'][...])
    g = jnp.tanh(g @ w['gen_W3'][...] + w['gen_b3'][...])
    bn = w['bn'][...]
    for n in range(_MAXN):
        p_n = g @ w['Wn%d' % n][...] + bn[n:n + 1, :]
        for c in range(_NC):
            sa_ref[c * _NB + n * _CG:c * _NB + (n + 1) * _CG, :] = \
                p_n[c * _CG:(c + 1) * _CG, :]

    for src, dst, l in ((sa_ref, sb_ref, 0), (sb_ref, sa_ref, 1)):
        wqkv = w['Wqkv%d' % l][...]
        for c2 in range(_NC // 2):
            qkv2 = (src[2 * c2 * _NB:(2 * c2 + 2) * _NB, :]
                    @ wqkv).astype(jnp.bfloat16)
            for half in range(2):
                c = 2 * c2 + half
                dst[c * _NB:(c + 1) * _NB, :] = _attn_chunk(
                    qkv2[half * _NB:(half + 1) * _NB, :], maskf)

    wqkv = w['Wqkv1'][...]
    for c2 in range(_NC // 2):
        qkv2 = (sa_ref[2 * c2 * _NB:(2 * c2 + 2) * _NB, :]
                @ wqkv).astype(jnp.bfloat16)
        for half in range(2):
            c = 2 * c2 + half
            emb = _attn_chunk(qkv2[half * _NB:(half + 1) * _NB, :], maskf)
            emb = _act(emb @ w['f_W1'][...])
            emb = _act(emb @ w['f_W2'][...])
            emb = emb @ w['f_W3'][...]
            for n in range(_MAXN):
                emb_ref[pl.Slice(c * _NB + n, _CG, _MAXN), :] = \
                    emb[n * _CG:(n + 1) * _CG, :]


def _edge_index_np(nb):
    i = np.arange(_MAXN)
    s0, d0 = np.meshgrid(i, i, indexing='ij')
    s0 = s0.reshape(-1)
    d0 = d0.reshape(-1)
    m = s0 != d0
    s0 = s0[m]
    d0 = d0[m]
    off = (np.arange(nb) * _MAXN)[:, None]
    src = (off + s0[None, :]).reshape(-1)
    dst = (off + d0[None, :]).reshape(-1)
    return np.stack([src, dst]).astype(np.int32)


def kernel(latent_vec, batch, params):
    nb = latent_vec.shape[0]
    n = nb * _MAXN
    grid = nb // _G
    scale = 1.0 / np.sqrt(float(_HID))

    p = dict(params)
    wp = {k: p[k] for k in ('nn_W1', 'nn_W2', 'nn_W3',
                            'gen_W1', 'gen_W2', 'gen_W3',
                            'f_W2', 'f_W3')}
    for k in ('nn_b1', 'nn_b2', 'nn_b3', 'gen_b1', 'gen_b2', 'gen_b3'):
        wp[k] = p[k].reshape(1, -1)
    for nn_ in range(_MAXN):
        wp['Wn%d' % nn_] = p['gen_Wn'][:, nn_ * _HID:(nn_ + 1) * _HID]
    wp['bn'] = p['gen_bn'].reshape(_MAXN, _HID)
    wp['Wqkv0'] = jnp.concatenate(
        [p['Wq0'] * scale, p['Wk0'], p['Wv0']], axis=1)
    wp['Wqkv1'] = jnp.concatenate(
        [p['Wq1'] * (scale / _HEADS), p['Wk1'] / _HEADS,
         p['Wv1'] / _HEADS], axis=1)
    wp['f_W1'] = p['f_W1'] / _HEADS
    r = np.arange(_NB)
    wp['maskf'] = jnp.asarray(
        (((r[:, None] % _CG) == (r[None, :] % _CG))
         & (r[:, None] != r[None, :])).astype(np.float32))

    ws = [wp[k] for k in _WEIGHT_NAMES]
    in_specs = [pl.BlockSpec((_G, _LATENT), lambda i: (i, 0))]
    for a in ws:
        in_specs.append(pl.BlockSpec(a.shape, lambda i: (0,) * a.ndim))

    emb, nn = pl.pallas_call(
        _fused_kernel,
        grid=(grid,),
        in_specs=in_specs,
        out_specs=[
            pl.BlockSpec((_G * _MAXN, _HID), lambda i: (i, 0)),
            pl.BlockSpec((_G, 1), lambda i: (i, 0)),
        ],
        out_shape=[
            jax.ShapeDtypeStruct((n, _HID), jnp.float32),
            jax.ShapeDtypeStruct((nb, 1), jnp.float32),
        ],
        scratch_shapes=[
            pltpu.VMEM((_NC * _NB, _HID), jnp.float32),
            pltpu.VMEM((_NC * _NB, _HID), jnp.float32),
        ],
    )(latent_vec, *ws)

    edge_index = jnp.asarray(_edge_index_np(nb))
    attr = jnp.zeros((edge_index.shape[1], _EDGE_DIM), jnp.float32)
    return emb, edge_index, attr, batch, nn.reshape(-1)


# speedup vs baseline: 1.3548x; 1.0369x over previous
import numpy as np
import jax
import jax.numpy as jnp
from jax.experimental import pallas as pl
from jax.experimental.pallas import tpu as pltpu

_LATENT = 128
_NODE_DIM = 32
_EDGE_DIM = 16
_HID = 32
_MAXN = 9
_HEADS = 4

_G = 256
_CG = 16
_NC = _G // _CG
_NB = _CG * _MAXN

_WEIGHT_NAMES = (
    'nn_W1', 'nn_b1', 'nn_W2', 'nn_b2', 'nn_W3', 'nn_b3',
    'gen_W1', 'gen_b1', 'gen_W2', 'gen_b2', 'gen_W3', 'gen_b3',
    'Wn0', 'Wn1', 'Wn2', 'Wn3', 'Wn4', 'Wn5', 'Wn6', 'Wn7', 'Wn8',
    'bn',
    'Wqkv0', 'Wqkv1',
    'f_W1', 'f_W2', 'f_W3',
    'maskf',
)


def _act(x):
    return jnp.where(x >= 0, x, 0.01 * x)


def _attn_chunk(qkv, maskf):
    acc = None
    for hd in range(_HEADS):
        sq = slice(hd * _HID, (hd + 1) * _HID)
        sk = slice(128 + hd * _HID, 128 + (hd + 1) * _HID)
        sv = slice(256 + hd * _HID, 256 + (hd + 1) * _HID)
        logits = jax.lax.dot_general(
            qkv[:, sq], qkv[:, sk], (((1,), (1,)), ((), ())),
            preferred_element_type=jnp.float32)
        ex = jnp.exp(logits) * maskf
        den = jnp.sum(ex, axis=1, keepdims=True)
        o = jax.lax.dot_general(
            ex.astype(jnp.bfloat16), qkv[:, sv], (((1,), (0,)), ((), ())),
            preferred_element_type=jnp.float32) / den
        acc = o if acc is None else acc + o
    return acc


def _fused_kernel(lat_ref, *refs):
    w = dict(zip(_WEIGHT_NAMES, refs[:len(_WEIGHT_NAMES)]))
    emb_ref, nn_ref, sa_ref, sb_ref = refs[len(_WEIGHT_NAMES):]
    lat = lat_ref[...]
    maskf = w['maskf'][...]

    h = _act(lat @ w['nn_W1'][...] + w['nn_b1'][...])
    h = _act(h @ w['nn_W2'][...] + w['nn_b2'][...])
    nn_ref[...] = h @ w['nn_W3'][...] + w['nn_b3'][...]

    g = jnp.tanh(lat @ w['gen_W1'][...] + w['gen_b1'][...])
    g = jnp.tanh(g @ w['gen_W2'][...] + w['gen_b2'][...])
    g = jnp.tanh(g @ w['gen_W3'][...] + w['gen_b3'][...])
    bn = w['bn'][...]
    for n in range(_MAXN):
        p_n = g @ w['Wn%d' % n][...] + bn[n:n + 1, :]
        for c in range(_NC):
            sa_ref[c * _NB + n * _CG:c * _NB + (n + 1) * _CG, :] = \
                p_n[c * _CG:(c + 1) * _CG, :]

    for src, dst, l in ((sa_ref, sb_ref, 0), (sb_ref, sa_ref, 1)):
        wqkv = w['Wqkv%d' % l][...]
        for c2 in range(_NC // 2):
            qkv2 = (src[2 * c2 * _NB:(2 * c2 + 2) * _NB, :]
                    @ wqkv).astype(jnp.bfloat16)
            for half in range(2):
                c = 2 * c2 + half
                dst[c * _NB:(c + 1) * _NB, :] = _attn_chunk(
                    qkv2[half * _NB:(half + 1) * _NB, :], maskf)

    wqkv = w['Wqkv1'][...]
    for c2 in range(_NC // 2):
        qkv2 = (sa_ref[2 * c2 * _NB:(2 * c2 + 2) * _NB, :]
                @ wqkv).astype(jnp.bfloat16)
        for half in range(2):
            c = 2 * c2 + half
            emb = _attn_chunk(qkv2[half * _NB:(half + 1) * _NB, :], maskf)
            emb = _act(emb @ w['f_W1'][...])
            emb = _act(emb @ w['f_W2'][...])
            emb = emb @ w['f_W3'][...]
            for n in range(_MAXN):
                emb_ref[pl.Slice(c * _NB + n, _CG, _MAXN), :] = \
                    emb[n * _CG:(n + 1) * _CG, :]


def _edge_index_np(nb):
    i = np.arange(_MAXN)
    s0, d0 = np.meshgrid(i, i, indexing='ij')
    s0 = s0.reshape(-1)
    d0 = d0.reshape(-1)
    m = s0 != d0
    s0 = s0[m]
    d0 = d0[m]
    off = (np.arange(nb) * _MAXN)[:, None]
    src = (off + s0[None, :]).reshape(-1)
    dst = (off + d0[None, :]).reshape(-1)
    return np.stack([src, dst]).astype(np.int32)


def kernel(latent_vec, batch, params):
    nb = latent_vec.shape[0]
    n = nb * _MAXN
    grid = nb // _G
    scale = 1.0 / np.sqrt(float(_HID))

    p = dict(params)
    wp = {k: p[k] for k in ('nn_W1', 'nn_W2', 'nn_W3',
                            'gen_W1', 'gen_W2', 'gen_W3',
                            'f_W2', 'f_W3')}
    for k in ('nn_b1', 'nn_b2', 'nn_b3', 'gen_b1', 'gen_b2', 'gen_b3'):
        wp[k] = p[k].reshape(1, -1)
    for nn_ in range(_MAXN):
        wp['Wn%d' % nn_] = p['gen_Wn'][:, nn_ * _HID:(nn_ + 1) * _HID]
    wp['bn'] = p['gen_bn'].reshape(_MAXN, _HID)
    wp['Wqkv0'] = jnp.concatenate(
        [p['Wq0'] * scale, p['Wk0'], p['Wv0']], axis=1)
    wp['Wqkv1'] = jnp.concatenate(
        [p['Wq1'] * (scale / _HEADS), p['Wk1'] / _HEADS,
         p['Wv1'] / _HEADS], axis=1)
    wp['f_W1'] = p['f_W1'] / _HEADS
    r = np.arange(_NB)
    wp['maskf'] = jnp.asarray(
        (((r[:, None] % _CG) == (r[None, :] % _CG))
         & (r[:, None] != r[None, :])).astype(np.float32))

    ws = [wp[k] for k in _WEIGHT_NAMES]
    in_specs = [pl.BlockSpec((_G, _LATENT), lambda i: (i, 0))]
    for a in ws:
        in_specs.append(pl.BlockSpec(a.shape, lambda i: (0,) * a.ndim))

    emb, nn = pl.pallas_call(
        _fused_kernel,
        grid=(grid,),
        in_specs=in_specs,
        out_specs=[
            pl.BlockSpec((_G * _MAXN, _HID), lambda i: (i, 0)),
            pl.BlockSpec((_G, 1), lambda i: (i, 0)),
        ],
        out_shape=[
            jax.ShapeDtypeStruct((n, _HID), jnp.float32),
            jax.ShapeDtypeStruct((nb, 1), jnp.float32),
        ],
        scratch_shapes=[
            pltpu.VMEM((_NC * _NB, _HID), jnp.float32),
            pltpu.VMEM((_NC * _NB, _HID), jnp.float32),
        ],
    )(latent_vec, *ws)

    edge_index = jnp.asarray(_edge_index_np(nb))
    attr = jnp.zeros((edge_index.shape[1], _EDGE_DIM), jnp.float32)
    return emb, edge_index, attr, batch, nn.reshape(-1)
